# baseline (speedup 1.0000x reference)
import functools

import jax
import jax.numpy as jnp
from jax.experimental import pallas as pl
from jax.experimental.pallas import tpu as pltpu

_BN = 4096
_BK = 8192
_BC = 1024


def _dcn_loss_kernel(emb_ref, cent_ref, out_ref, acc_ref, a_ref, *, inv_n):
    i = pl.program_id(0)
    j = pl.program_id(1)
    nj = pl.num_programs(1)

    @pl.when(jnp.logical_and(i == 0, j == 0))
    def _build_a():
        c = cent_ref[...]
        c_sq = jnp.sum(c * c, axis=1, keepdims=True)
        a_f32 = jnp.concatenate([c, -0.5 * c_sq], axis=1)
        a_ref[...] = a_f32.T.astype(jnp.bfloat16)

    x = emb_ref[...]
    a = a_ref[...]
    g = jnp.dot(x, a, preferred_element_type=jnp.float32)
    part = jnp.max(g, axis=1, keepdims=True)

    @pl.when(j == 0)
    def _init():
        acc_ref[...] = part

    @pl.when(j != 0)
    def _fold():
        acc_ref[...] = jnp.maximum(acc_ref[...], part)

    @pl.when(j == nj - 1)
    def _finish():
        xf = x.astype(jnp.float32)
        x_sq = jnp.sum(xf * xf, axis=1, keepdims=True) - 1.0
        s = jnp.sum(x_sq - 2.0 * acc_ref[...]) * inv_n

        @pl.when(i == 0)
        def _first():
            out_ref[0, 0] = s

        @pl.when(i != 0)
        def _rest():
            out_ref[0, 0] = out_ref[0, 0] + s


def kernel(embedded, centers):
    n, d = embedded.shape
    k, _ = centers.shape
    ni, nj = n // _BN, k // _BK

    emb_aug = jnp.concatenate(
        [embedded, jnp.ones((n, 1), jnp.float32)], axis=1
    ).astype(jnp.bfloat16)

    total = pl.pallas_call(
        functools.partial(_dcn_loss_kernel, inv_n=1.0 / n),
        grid=(ni, nj),
        in_specs=[
            pl.BlockSpec((_BN, d + 1), lambda i, j: (i, 0)),
            pl.BlockSpec((k, d), lambda i, j: (0, 0)),
        ],
        out_specs=pl.BlockSpec(memory_space=pltpu.SMEM),
        out_shape=jax.ShapeDtypeStruct((1, 1), jnp.float32),
        scratch_shapes=[pltpu.VMEM((_BN, 1), jnp.float32),
                        pltpu.VMEM((d + 1, k), jnp.bfloat16)],
        compiler_params=pltpu.CompilerParams(
            dimension_semantics=("arbitrary", "arbitrary")
        ),
    )(emb_aug, centers)
    return total[0, 0]


# speedup vs baseline: 1.0689x; 1.0689x over previous
import functools

import jax
import jax.numpy as jnp
from jax.experimental import pallas as pl
from jax.experimental.pallas import tpu as pltpu

_BN = 4096


def _dcn_loss_kernel(emb_ref, a_ref, out_ref, *, inv_n):
    i = pl.program_id(0)

    x = emb_ref[...]
    a = a_ref[...]
    g = jnp.dot(x, a, preferred_element_type=jnp.float32)
    part = jnp.max(g, axis=1, keepdims=True)

    xf = x.astype(jnp.float32)
    x_sq = jnp.sum(xf * xf, axis=1, keepdims=True) - 1.0
    s = jnp.sum(x_sq - 2.0 * part) * inv_n

    @pl.when(i == 0)
    def _first():
        out_ref[0, 0] = s

    @pl.when(i != 0)
    def _rest():
        out_ref[0, 0] = out_ref[0, 0] + s


def kernel(embedded, centers):
    n, d = embedded.shape
    k, _ = centers.shape
    ni = n // _BN

    emb_aug = jnp.concatenate(
        [embedded, jnp.ones((n, 1), jnp.float32)], axis=1
    ).astype(jnp.bfloat16)
    c_sq = jnp.sum(centers * centers, axis=1)
    a_mat = jnp.concatenate(
        [centers.T, -0.5 * c_sq[None, :]], axis=0
    ).astype(jnp.bfloat16)

    total = pl.pallas_call(
        functools.partial(_dcn_loss_kernel, inv_n=1.0 / n),
        grid=(ni,),
        in_specs=[
            pl.BlockSpec((_BN, d + 1), lambda i: (i, 0)),
            pl.BlockSpec((d + 1, k), lambda i: (0, 0)),
        ],
        out_specs=pl.BlockSpec(memory_space=pltpu.SMEM),
        out_shape=jax.ShapeDtypeStruct((1, 1), jnp.float32),
        compiler_params=pltpu.CompilerParams(
            dimension_semantics=("arbitrary",)
        ),
    )(emb_aug, a_mat)
    return total[0, 0]
